# Initial kernel scaffold; baseline (speedup 1.0000x reference)
import jax
import jax.numpy as jnp
from jax.experimental import pallas as pl


def kernel(token_ids, type_ids, token_table, type_table, pos_table, ln_gamma, ln_beta):
    raise NotImplementedError("write your pallas kernel here")


# speedup vs baseline: 1.4085x; 1.4085x over previous
import functools

import jax
import jax.numpy as jnp
from jax import lax
from jax.experimental import pallas as pl
from jax.experimental.pallas import tpu as pltpu
from jax.experimental.pallas import tpu_sc as plsc

HIDDEN = 64
SEQ = 200
CHUNK = 800
SUB = 100
NSUB = CHUNK // SUB
NWORKERS = 32
EPS = 1e-5


def _rsqrt16(x):
    i = lax.bitcast_convert_type(x, jnp.int32)
    y = lax.bitcast_convert_type(jnp.int32(0x5F3759DF) - (i >> 1), jnp.float32)
    hx = x * (-0.5)
    for _ in range(3):
        y = y * (1.5 + hx * y * y)
    return y


@functools.partial(jax.jit, static_argnames=("batch", "seq"))
def _run(token_ids, type_ids, token_table, type_table, pos_table, ln_gamma,
         ln_beta, *, batch, seq):
    flat = batch * seq
    tpw = flat // NWORKERS
    nchunk = tpw // CHUNK

    tok2 = token_ids.reshape(flat // SUB, SUB).astype(jnp.int32)
    typf = type_ids.reshape(flat).astype(jnp.int32)

    mesh = plsc.VectorSubcoreMesh(core_axis_name="c", subcore_axis_name="s")

    @functools.partial(
        pl.kernel,
        mesh=mesh,
        compiler_params=pltpu.CompilerParams(use_tc_tiling_on_sc=False),
        out_type=jax.ShapeDtypeStruct((flat, HIDDEN), jnp.float32),
        scratch_types=[
            pltpu.VMEM((NSUB, SUB), jnp.int32),
            pltpu.VMEM((NSUB, SUB), jnp.int32),
            pltpu.VMEM((CHUNK + 16,), jnp.int32),
            pltpu.VMEM((CHUNK + 16,), jnp.int32),
            pltpu.VMEM((CHUNK, HIDDEN), jnp.float32),
            pltpu.VMEM((CHUNK, HIDDEN), jnp.float32),
            pltpu.VMEM((SEQ, HIDDEN), jnp.float32),
            pltpu.VMEM((4, HIDDEN), jnp.float32),
            pltpu.VMEM((HIDDEN,), jnp.float32),
            pltpu.VMEM((HIDDEN,), jnp.float32),
            pltpu.SemaphoreType.DMA,
            pltpu.SemaphoreType.DMA,
            pltpu.SemaphoreType.DMA,
            pltpu.SemaphoreType.DMA,
        ],
    )
    def sc_kernel(tok_hbm, typ_hbm, table_hbm, type_t_hbm, pos_hbm, g_hbm,
                  b_hbm, out_hbm, idx0, idx1, tix0, tix1, rows0, rows1, posb,
                  typb, gv, bv, gsem0, gsem1, ssem0, ssem1):
        wid = lax.axis_index("s") * 2 + lax.axis_index("c")
        base = wid * tpw

        pltpu.sync_copy(pos_hbm.at[pl.ds(0, SEQ)], posb)
        pltpu.sync_copy(type_t_hbm, typb)
        pltpu.sync_copy(g_hbm, gv)
        pltpu.sync_copy(b_hbm, bv)

        g = [gv[pl.ds(k * 16, 16)] for k in range(4)]
        b = [bv[pl.ds(k * 16, 16)] for k in range(4)]
        lanes = lax.iota(jnp.int32, 16)
        bfly = [lanes ^ sh for sh in (1, 2, 4, 8)]

        idx = (idx0, idx1)
        tix = (tix0, tix1)
        rows = (rows0, rows1)
        gsem = (gsem0, gsem1)
        ssem = (ssem0, ssem1)

        def start_gather(c, bi):
            cbase = pl.multiple_of(base + c * CHUNK, CHUNK)
            pltpu.sync_copy(
                tok_hbm.at[pl.ds(pl.multiple_of(cbase // SUB, NSUB), NSUB)],
                idx[bi])
            pltpu.sync_copy(typ_hbm.at[pl.ds(cbase, CHUNK)],
                            tix[bi].at[pl.ds(0, CHUNK)])
            for j in range(NSUB):
                pltpu.async_copy(
                    table_hbm.at[idx[bi].at[j]],
                    rows[bi].at[pl.ds(j * SUB, SUB)],
                    gsem[bi],
                )

        def wait_gather(bi):
            pltpu.make_async_copy(
                out_hbm.at[pl.ds(0, CHUNK)], rows[bi], gsem[bi]
            ).wait()

        def start_scatter(c, bi):
            pltpu.async_copy(
                rows[bi],
                out_hbm.at[pl.ds(pl.multiple_of(base + c * CHUNK, CHUNK),
                                 CHUNK)],
                ssem[bi],
            )

        def wait_scatter(bi):
            pltpu.make_async_copy(
                rows[bi], out_hbm.at[pl.ds(0, CHUNK)], ssem[bi]
            ).wait()

        def compute_chunk(bi):
            rbuf = rows[bi]
            tbuf = tix[bi]

            def tok_body(i, p):
                tid = tbuf[pl.ds(i, 16)][0]
                y = [
                    rbuf[i, pl.ds(k * 16, 16)]
                    + typb[tid, pl.ds(k * 16, 16)]
                    + posb[p, pl.ds(k * 16, 16)]
                    for k in range(4)
                ]
                s = (y[0] + y[1]) + (y[2] + y[3])
                q = (y[0] * y[0] + y[1] * y[1]) + (y[2] * y[2] + y[3] * y[3])
                for perm in bfly:
                    s = s + s.at[perm].get(mode="promise_in_bounds")
                    q = q + q.at[perm].get(mode="promise_in_bounds")
                mv = s * (1.0 / HIDDEN)
                var = q * (1.0 / HIDDEN) - mv * mv
                inv = _rsqrt16(var + EPS)
                for k in range(4):
                    rbuf[i, pl.ds(k * 16, 16)] = (y[k] - mv) * inv * g[k] + b[k]
                return jnp.where(p == SEQ - 1, 0, p + 1)

            lax.fori_loop(0, CHUNK, tok_body, jnp.int32(0), unroll=4)

        start_gather(0, 0)

        def outer(co, _):
            for bstat in range(2):
                c = co * 2 + bstat
                nb = 1 - bstat

                @pl.when(c >= 1)
                def _():
                    wait_scatter(nb)

                @pl.when(c + 1 < nchunk)
                def _():
                    start_gather(c + 1, nb)

                wait_gather(bstat)
                compute_chunk(bstat)
                start_scatter(c, bstat)
            return 0

        lax.fori_loop(0, nchunk // 2, outer, 0)
        wait_scatter(1)

    out = sc_kernel(tok2, typf, token_table, type_table, pos_table, ln_gamma,
                    ln_beta)
    return out.reshape(batch, seq, HIDDEN)


def kernel(token_ids, type_ids, token_table, type_table, pos_table, ln_gamma,
           ln_beta):
    batch, seq = token_ids.shape
    return _run(token_ids, type_ids, token_table, type_table, pos_table,
                ln_gamma, ln_beta, batch=batch, seq=seq)
